# Initial kernel scaffold; baseline (speedup 1.0000x reference)
import jax
import jax.numpy as jnp
from jax.experimental import pallas as pl


def kernel(E_u_0, E_i_0, adj_rows, adj_cols, adj_vals, u_mul_s, v_mul_s, ut, vt, W0, W1, uids, iids, pos, neg, active_list, eps_1, eps_2, eps_3):
    raise NotImplementedError("write your pallas kernel here")


# speedup vs baseline: 3.9629x; 3.9629x over previous
import functools

import jax
import jax.numpy as jnp
from jax import lax
from jax.experimental import pallas as pl
from jax.experimental.pallas import tpu as pltpu
from jax.experimental.pallas import tpu_sc as plsc

N_U = 50000
N_I = 50000
D = 64
Q = 5
NNZ = 800000
B = 4096
TEMP = 0.2
LAMBDA_1 = 0.2

NC = 2
NS = 16
H = 32
CH = 128
NSTR = 5
CHUNK = CH * NSTR
FL = 80

_SC_PARAMS = pltpu.CompilerParams(use_tc_tiling_on_sc=False)
@functools.lru_cache(maxsize=None)
def _mesh():
    return plsc.VectorSubcoreMesh(core_axis_name="c", subcore_axis_name="s",
                                  num_cores=NC, num_subcores=NS)


@functools.lru_cache(maxsize=None)
def _make_spmm(n_dst, n_src, nnz, tbl_interleaved, prev_interleaved):
    assert nnz % CHUNK == 0
    nchunks = nnz // CHUNK
    fl = FL
    assert n_dst % fl == 0
    nblk = n_dst // fl

    def body(src_hbm, dst_hbm, val_hbm, table_hbm, prev_hbm, zout_hbm, eout_hbm,
             acc_sh, sidx, didx, vals, gbuf, fbuf, ebuf, sem):
        c = lax.axis_index("c")
        s = lax.axis_index("s")
        coff_dst = c * n_dst

        def zrow(r, _):
            z16 = jnp.zeros((16,), jnp.float32)
            fbuf[r, pl.ds(0, 16)] = z16
            fbuf[r, pl.ds(16, 16)] = z16
            return 0
        lax.fori_loop(0, fl, zrow, 0)

        def zcopy(j, _):
            k = j * NS + s

            @pl.when(k < nblk)
            def _():
                pltpu.sync_copy(fbuf, acc_sh.at[pl.ds(k * fl, fl)])
            return 0
        lax.fori_loop(0, (nblk + NS - 1) // NS, zcopy, 0)
        plsc.subcore_barrier()

        def chunk_body(j, _):
            k = j * NS + s

            @pl.when(k < nchunks)
            def _():
                base = k * CHUNK
                pltpu.sync_copy(src_hbm.at[pl.ds(base, CHUNK)], sidx)
                pltpu.sync_copy(val_hbm.at[pl.ds(base, CHUNK)], vals)
                for j2 in range(NSTR):
                    pltpu.sync_copy(dst_hbm.at[pl.ds(base + j2 * CH, CH)],
                                    didx.at[j2])

                def shift(i, _):
                    v = sidx[pl.ds(i * 16, 16)]
                    if tbl_interleaved:
                        sidx[pl.ds(i * 16, 16)] = v * 2 + c
                    else:
                        sidx[pl.ds(i * 16, 16)] = v + c * n_src
                    return 0
                lax.fori_loop(0, CHUNK // 16, shift, 0)

                for j2 in range(NSTR):
                    pltpu.sync_copy(table_hbm.at[sidx.at[pl.ds(j2 * CH, CH)]],
                                    gbuf.at[pl.ds(j2 * CH, CH)])

                def scale(i, _):
                    vv = vals[pl.ds(i * 16, 16)]
                    for e in range(16):
                        spl = vv.at[jnp.full((16,), e, jnp.int32)].get(
                            mode="promise_in_bounds")
                        r = i * 16 + e
                        gbuf[r, pl.ds(0, 16)] = gbuf[r, pl.ds(0, 16)] * spl
                        gbuf[r, pl.ds(16, 16)] = gbuf[r, pl.ds(16, 16)] * spl
                    return 0
                lax.fori_loop(0, CHUNK // 16, scale, 0)

                for j2 in range(NSTR):
                    pltpu.sync_copy(gbuf.at[pl.ds(j2 * CH, CH)],
                                    acc_sh.at[didx.at[j2]], add=True)
            return 0
        lax.fori_loop(0, (nchunks + NS - 1) // NS, chunk_body, 0)
        plsc.subcore_barrier()

        def flush(j, _):
            k = j * NS + s

            @pl.when(k < nblk)
            def _():
                row0 = k * fl
                pltpu.sync_copy(acc_sh.at[pl.ds(row0, fl)], fbuf)
                if prev_interleaved:
                    def pidx(i, _):
                        base16 = row0 + i * 16
                        didx[0, pl.ds(i * 16, 16)] = (
                            (base16 + lax.iota(jnp.int32, 16)) * 2 + c)
                        return 0
                    lax.fori_loop(0, fl // 16, pidx, 0)
                    pltpu.sync_copy(
                        prev_hbm.at[didx.at[0, pl.ds(0, fl)]], ebuf)
                else:
                    pltpu.sync_copy(
                        prev_hbm.at[pl.ds(coff_dst + row0, fl)], ebuf)

                def actrow(r, _):
                    for hh in range(2):
                        x = fbuf[r, pl.ds(hh * 16, 16)]
                        a = jnp.maximum(x, 0.5 * x)
                        fbuf[r, pl.ds(hh * 16, 16)] = a
                        ebuf[r, pl.ds(hh * 16, 16)] = (
                            ebuf[r, pl.ds(hh * 16, 16)] + a)
                    return 0
                lax.fori_loop(0, fl, actrow, 0)
                pltpu.sync_copy(fbuf, zout_hbm.at[pl.ds(coff_dst + row0, fl)])
                pltpu.sync_copy(ebuf, eout_hbm.at[pl.ds(coff_dst + row0, fl)])
            return 0
        lax.fori_loop(0, (nblk + NS - 1) // NS, flush, 0)

    f32 = jnp.float32
    return pl.kernel(
        body,
        out_type=[jax.ShapeDtypeStruct((2 * n_dst, H), f32),
                  jax.ShapeDtypeStruct((2 * n_dst, H), f32)],
        mesh=_mesh(),
        compiler_params=_SC_PARAMS,
        scratch_types=[
            pltpu.VMEM_SHARED((n_dst, H), f32),
            pltpu.VMEM((CHUNK,), jnp.int32),
            pltpu.VMEM((NSTR, CH), jnp.int32),
            pltpu.VMEM((CHUNK,), f32),
            pltpu.VMEM((CHUNK, H), f32),
            pltpu.VMEM((fl, H), f32),
            pltpu.VMEM((fl, H), f32),
            pltpu.SemaphoreType.DMA,
        ],
    )


def _make_batch_gather():
    f32 = jnp.float32
    nchunk = B // CH

    def body(zu1s, zu2s, zi1s, zi2s, eu0, ei0, mus16, mvs16,
             uids, iids, pos, neg,
             zu1b, zu2b, zi1bi, zi2bi, zi1bp, zi2bp, zi1bn, zi2bn,
             eu0b, ei0bp, ei0bn, musb, mvsb,
             ridx, sidx, gb32, gb64, gb16, sem):
        c = lax.axis_index("c")
        s = lax.axis_index("s")

        idx_refs = [uids, iids, pos, neg]
        for a in range(4):
            for kk in range(2):
                k = s * 2 + kk
                pltpu.sync_copy(idx_refs[a].at[pl.ds(k * CH, CH)],
                                ridx.at[a * 2 + kk])
                for i in range(CH // 16):
                    v = ridx[a * 2 + kk, pl.ds(i * 16, 16)]
                    sidx[a * 2 + kk, pl.ds(i * 16, 16)] = v + c * N_U

        specs32 = [(zu1s, 0, zu1b), (zu2s, 0, zu2b),
                   (zi1s, 1, zi1bi), (zi2s, 1, zi2bi),
                   (zi1s, 2, zi1bp), (zi2s, 2, zi2bp),
                   (zi1s, 3, zi1bn), (zi2s, 3, zi2bn)]
        for tbl, a, out in specs32:
            for kk in range(2):
                k = s * 2 + kk
                pltpu.sync_copy(tbl.at[sidx.at[a * 2 + kk]], gb32)
                pltpu.sync_copy(gb32, out.at[pl.ds(c * B + k * CH, CH)])

        specs64 = [(eu0, 0, eu0b), (ei0, 2, ei0bp), (ei0, 3, ei0bn)]
        for tbl, a, out in specs64:
            kk = c
            k = s * 2 + kk
            pltpu.sync_copy(tbl.at[ridx.at[a * 2 + kk]], gb64)
            pltpu.sync_copy(gb64, out.at[pl.ds(k * CH, CH)])

        @pl.when(c == 0)
        def _():
            specs16 = [(mus16, 0, musb), (mvs16, 1, mvsb)]
            for tbl, a, out in specs16:
                for kk in range(2):
                    k = s * 2 + kk
                    pltpu.sync_copy(tbl.at[ridx.at[a * 2 + kk]], gb16)
                    pltpu.sync_copy(gb16, out.at[pl.ds(k * CH, CH)])

    sds = jax.ShapeDtypeStruct
    return pl.kernel(
        body,
        out_type=[sds((2 * B, H), f32)] * 8 + [sds((B, 64), f32)] * 3
                 + [sds((B, 16), f32)] * 2,
        mesh=_mesh(),
        compiler_params=_SC_PARAMS,
        scratch_types=[
            pltpu.VMEM((8, CH), jnp.int32),
            pltpu.VMEM((8, CH), jnp.int32),
            pltpu.VMEM((CH, H), f32),
            pltpu.VMEM((CH, 64), f32),
            pltpu.VMEM((CH, 16), f32),
            pltpu.SemaphoreType.DMA,
        ],
    )


_RB = 2000


def _qmat64(f_pad_t, tbl):
    n = tbl.shape[0]
    nb = n // _RB

    def body(f_ref, t_ref, o_ref):
        j = pl.program_id(0)

        @pl.when(j == 0)
        def _():
            o_ref[...] = jnp.zeros_like(o_ref)
        o_ref[...] += lax.dot_general(
            f_ref[...], t_ref[...], (((0,), (0,)), ((), ())),
            preferred_element_type=jnp.float32)

    return pl.pallas_call(
        body,
        grid=(nb,),
        in_specs=[pl.BlockSpec((_RB, 8), lambda j: (j, 0)),
                  pl.BlockSpec((_RB, 64), lambda j: (j, 0))],
        out_specs=pl.BlockSpec((8, 64), lambda j: (0, 0)),
        out_shape=jax.ShapeDtypeStruct((8, 64), jnp.float32),
    )(f_pad_t, tbl)


def _qmat32(f_pad_t, tbl_s):
    n = tbl_s.shape[0] // 2
    nb = n // _RB

    def body(f_ref, t_ref, o_ref):
        j = pl.program_id(1)

        @pl.when(j == 0)
        def _():
            o_ref[...] = jnp.zeros_like(o_ref)
        o_ref[...] += lax.dot_general(
            f_ref[...], t_ref[...], (((0,), (0,)), ((), ())),
            preferred_element_type=jnp.float32)[None]

    return pl.pallas_call(
        body,
        grid=(2, nb),
        in_specs=[pl.BlockSpec((_RB, 8), lambda h, j: (j, 0)),
                  pl.BlockSpec((_RB, 32), lambda h, j: (h * nb + j, 0))],
        out_specs=pl.BlockSpec((1, 8, 32), lambda h, j: (h, 0, 0)),
        out_shape=jax.ShapeDtypeStruct((2, 8, 32), jnp.float32),
    )(f_pad_t, tbl_s)


_NSB = 512


def _l2n(x):
    nrm = jnp.sqrt(jnp.sum(x * x, axis=1, keepdims=True))
    return x / jnp.maximum(nrm, 1e-12)


def _act(x):
    return jnp.where(x >= 0, x, 0.5 * x)


def _hyper(musb, mvsb, qmats, ws):
    def body(mus_r, mvs_r, qm_r, w_r, o_ref):
        for l in (1, 2):
            for side in (0, 1):
                idx = 2 * (l - 1) + side
                m = mus_r[...] if side == 0 else mvs_r[...]
                h = jnp.dot(_l2n(_act(jnp.dot(
                    m, qm_r[idx], preferred_element_type=jnp.float32))),
                    w_r[l - 1], preferred_element_type=jnp.float32)
                o_ref[idx] = h

    return pl.pallas_call(
        body,
        out_shape=jax.ShapeDtypeStruct((4, B, 64), jnp.float32),
    )(musb, mvsb, qmats, ws)


def _losses(zu1b, zu2b, zi1bi, zi2bi, eu0b, ei0bp, ei0bn,
            zi1bp, zi2bp, zi1bn, zi2bn, musb, mvsb, qmats, ws, masks):
    hyps = _hyper(musb, mvsb, qmats, ws)
    nrb = B // _NSB

    def body(zu1_r, zu2_r, zi1i_r, zi2i_r, eu0_r, ei0p_r, ei0n_r,
             zi1p_r, zi2p_r, zi1n_r, zi2n_r, hyps_r, mk_r, o_ref):
        jb = pl.program_id(0)

        @pl.when(jb == 0)
        def _():
            o_ref[...] = jnp.zeros_like(o_ref)

        row0 = pl.multiple_of(jb * _NSB, _NSB)
        loss_s = jnp.float32(0.0)
        zblks = [zu1_r[...], zi1i_r[...], zu2_r[...], zi2i_r[...]]
        for idx in range(4):
            g = _l2n(zblks[idx])
            hrow = hyps_r[idx, pl.ds(row0, _NSB), :]
            ps = jnp.exp(jnp.sum(g * hrow, axis=1) / TEMP)
            ns = jnp.zeros((_NSB,), jnp.float32)
            for cb in range(nrb):
                hblk = hyps_r[idx, pl.ds(cb * _NSB, _NSB), :]
                sc = lax.dot_general(g, hblk, (((1,), (1,)), ((), ())),
                                     preferred_element_type=jnp.float32)
                ns = ns + jnp.sum(jnp.exp(sc / TEMP), axis=1)
            mask = mk_r[idx, :]
            loss_s = loss_s + jnp.sum(
                -jnp.log(ps / (ns + 1e-08) + 1e-08) * mask)

        eub = 3.0 * eu0_r[...] + 2.0 * zu1_r[...] + zu2_r[...]
        eip = 3.0 * ei0p_r[...] + 2.0 * zi1p_r[...] + zi2p_r[...]
        ein = 3.0 * ei0n_r[...] + 2.0 * zi1n_r[...] + zi2n_r[...]
        pscore = jnp.sum(eub * eip, axis=1)
        nscore = jnp.sum(eub * ein, axis=1)
        loss_r = jnp.sum(jnp.maximum(1.0 - pscore + nscore, 0.0))
        row = jnp.concatenate([loss_s[None], loss_r[None],
                               jnp.zeros((126,), jnp.float32)])
        o_ref[...] += jnp.broadcast_to(row, (8, 128))

    bspec = lambda: pl.BlockSpec((_NSB, 64), lambda j: (j, 0))
    out = pl.pallas_call(
        body,
        grid=(nrb,),
        in_specs=[bspec(), bspec(), bspec(), bspec(), bspec(), bspec(),
                  bspec(), bspec(), bspec(), bspec(), bspec(),
                  pl.BlockSpec((4, B, 64), lambda j: (0, 0, 0)),
                  pl.BlockSpec((4, _NSB), lambda j: (0, j))],
        out_specs=pl.BlockSpec((8, 128), lambda j: (0, 0)),
        out_shape=jax.ShapeDtypeStruct((8, 128), jnp.float32),
    )(zu1b, zu2b, zi1bi, zi2bi, eu0b, ei0bp, ei0bn,
      zi1bp, zi2bp, zi1bn, zi2bn, hyps, masks)
    ls = LAMBDA_1 * out[0, 0]
    loss_r = out[0, 1]
    return loss_r + ls, loss_r, ls


def _merge(x2b):
    return jnp.concatenate([x2b[:B], x2b[B:]], axis=1)


def kernel(E_u_0, E_i_0, adj_rows, adj_cols, adj_vals, u_mul_s, v_mul_s, ut, vt,
           W0, W1, uids, iids, pos, neg, active_list, eps_1, eps_2, eps_3):
    f32 = jnp.float32
    eu0i = E_u_0.reshape(2 * N_U, H)
    ei0i = E_i_0.reshape(2 * N_I, H)

    spmm_l1 = _make_spmm(N_U, N_I, NNZ, True, True)
    spmm_l2 = _make_spmm(N_U, N_I, NNZ, False, False)

    zu1s, eu1s = spmm_l1(adj_cols, adj_rows, adj_vals, ei0i, eu0i)
    zi1s, ei1s = spmm_l1(adj_rows, adj_cols, adj_vals, eu0i, ei0i)
    zu2s, _ = spmm_l2(adj_cols, adj_rows, adj_vals, ei1s, eu1s)
    zi2s, _ = spmm_l2(adj_rows, adj_cols, adj_vals, eu1s, ei1s)

    mus16 = jnp.pad(u_mul_s, ((0, 0), (0, 16 - Q)))
    mvs16 = jnp.pad(v_mul_s, ((0, 0), (0, 16 - Q)))
    (zu1b2, zu2b2, zi1bi2, zi2bi2, zi1bp2, zi2bp2, zi1bn2, zi2bn2,
     eu0b, ei0bp, ei0bn, musb, mvsb) = _make_batch_gather()(
        zu1s, zu2s, zi1s, zi2s, E_u_0, E_i_0, mus16, mvs16,
        uids, iids, pos, neg)

    vt8 = jnp.pad(vt.T, ((0, 0), (0, 8 - Q)))
    ut8 = jnp.pad(ut.T, ((0, 0), (0, 8 - Q)))
    vt_ei1 = _qmat64(vt8, E_i_0)
    ut_eu1 = _qmat64(ut8, E_u_0)
    v2 = _qmat32(vt8, ei1s)
    u2 = _qmat32(ut8, eu1s)
    vt_ei2 = jnp.concatenate([v2[0], v2[1]], axis=1)
    ut_eu2 = jnp.concatenate([u2[0], u2[1]], axis=1)
    pad8 = lambda m: jnp.pad(m, ((0, 8), (0, 0)))
    qmats = jnp.stack([pad8(vt_ei1), pad8(ut_eu1),
                       pad8(vt_ei2), pad8(ut_eu2)])

    ws = jnp.stack([W0, W1])

    mkey = jax.random.key(42)
    masks = jnp.stack([
        (jax.random.uniform(jax.random.fold_in(mkey, t), (B,)) > 0.5)
        .astype(f32) for t in (2, 3, 4, 5)])

    loss, loss_r, ls = _losses(
        _merge(zu1b2), _merge(zu2b2), _merge(zi1bi2), _merge(zi2bi2),
        eu0b, ei0bp, ei0bn,
        _merge(zi1bp2), _merge(zi2bp2), _merge(zi1bn2), _merge(zi2bn2),
        musb, mvsb, qmats, ws, masks)
    return loss, loss_r, ls
